# baseline (speedup 1.0000x reference)
import functools

import jax
import jax.numpy as jnp
from jax.experimental import pallas as pl
from jax.experimental.pallas import tpu as pltpu

IN_F = 2048
HID_F = 4096
OUT_F = 2048
N_TOK = 8192
K_TOP = max(1, int(HID_F * 0.15))

T1 = 128
T2 = 256


def _quant_kernel(w_ref, s_ref, o_ref):
    s = s_ref[0, 0]
    w = w_ref[...]
    o_ref[...] = jnp.round(jnp.clip(w / s, -1.0, 1.0)) * s


def _sortable_i32(z):
    m = jax.lax.bitcast_convert_type(z, jnp.int32)
    neg = jnp.bitwise_xor(jnp.bitwise_not(m), jnp.int32(-2147483648))
    return jnp.where(m >= 0, m, neg)


def _topk_mask_deep(z, deep_ref):
    mz = jnp.mean(z, axis=1, keepdims=True)
    sz = jnp.sqrt(jnp.maximum(
        jnp.mean(z * z, axis=1, keepdims=True) - mz * mz, 0.0))
    sz = sz * 1.00001 + 1e-30
    lo0 = mz - 0.43 * sz
    hi0 = mz + 2.50 * sz

    def vbody(_, carry):
        lo, hi, clo, chi = carry
        w = hi - lo
        denom = jnp.maximum((clo - chi).astype(jnp.float32), 1.0)
        frac = (clo - K_TOP).astype(jnp.float32) / denom
        mid = lo + w * jnp.clip(frac, 0.04, 0.96)
        cnt = jnp.sum((z >= mid).astype(jnp.int32), axis=1, keepdims=True)
        ge = cnt >= K_TOP
        eq = cnt == K_TOP
        lo = jnp.where(ge, mid, lo)
        clo = jnp.where(ge, cnt, clo)
        hi = jnp.where(eq, mid, jnp.where(ge, hi, mid))
        chi = jnp.where(eq, cnt, jnp.where(ge, chi, cnt))
        return lo, hi, clo, chi

    t_blk = z.shape[0]
    c_lo0 = jnp.full((t_blk, 1), 639, dtype=jnp.int32)
    c_hi0 = jnp.full((t_blk, 1), 564, dtype=jnp.int32)
    carry = (lo0, hi0, c_lo0, c_hi0)
    for _ in range(6):
        carry = vbody(0, carry)
    lof, hif, _, _ = carry

    s = _sortable_i32(z)
    klo0 = _sortable_i32(lof)
    khi0 = _sortable_i32(hif)

    def kcond(carry):
        i, klo, khi = carry
        return jnp.logical_and(i < 32, jnp.any(khi > klo + 1))

    def kpass(klo, khi):
        mid = jnp.bitwise_and(klo, khi) + jnp.right_shift(
            jnp.bitwise_xor(klo, khi), 1)
        cnt = jnp.sum((s >= mid).astype(jnp.int32), axis=1, keepdims=True)
        ge = cnt >= K_TOP
        eq = cnt == K_TOP
        klo = jnp.where(ge, mid, klo)
        khi = jnp.where(eq, mid + 1, jnp.where(ge, khi, mid))
        return klo, khi

    def kbody(carry):
        i, klo, khi = carry
        klo, khi = kpass(klo, khi)
        klo, khi = kpass(klo, khi)
        return i + 2, klo, khi

    for _ in range(10):
        klo0, khi0 = kpass(klo0, khi0)
    _, klo, _ = jax.lax.while_loop(kcond, kbody, (0, klo0, khi0))
    mask = (s >= klo).astype(jnp.float32)
    deep_ref[...] = z * mask


def _fwd1_kernel(x_ref, w_ref, b1_ref, g1_ref, be1_ref, g2_ref, be2_ref,
                 res_ref, deep_ref, zscr_ref):
    i = pl.program_id(0)
    p = jax.lax.rem(i, 2)

    @pl.when(i == 0)
    def _init():
        zscr_ref[...] = jnp.zeros_like(zscr_ref)

    x = x_ref[...]
    h = jax.lax.dot_general(x, w_ref[...], (((1,), (1,)), ((), ())),
                            preferred_element_type=jnp.float32)
    h = h + b1_ref[...]
    h = jnp.maximum(h, 0.0)
    mu = jnp.mean(h, axis=1, keepdims=True)
    d = h - mu
    var = jnp.mean(d * d, axis=1, keepdims=True)
    res = d / jnp.sqrt(var + 1e-5) * g1_ref[...] + be1_ref[...]
    mu2 = jnp.mean(res, axis=1, keepdims=True)
    d2 = res - mu2
    var2 = jnp.mean(d2 * d2, axis=1, keepdims=True)
    z = d2 / jnp.sqrt(var2 + 1e-5) * g2_ref[...] + be2_ref[...]

    z_prev = zscr_ref[pl.ds(1 - p, 1), :, :][0]
    _topk_mask_deep(z_prev, deep_ref)

    res_ref[...] = res
    zscr_ref[pl.ds(p, 1), :, :] = z[None]


def _fwd2_kernel(d_ref, w_ref, b2_ref, vth_ref,
                 out_ref, spk_ref, mem_ref, sp_ref):
    mem = jax.lax.dot_general(d_ref[...], w_ref[...], (((1,), (1,)), ((), ())),
                              preferred_element_type=jnp.float32)
    mem = mem + b2_ref[...]
    vth = vth_ref[...]
    sp = jax.nn.sigmoid((mem - vth) / 0.1)
    hard = (mem > vth).astype(jnp.float32)
    mem_ref[...] = mem
    sp_ref[...] = sp
    spk_ref[...] = hard
    out_ref[...] = mem * hard


@jax.jit
def kernel(x_input, W1, b1, ln1_g, ln1_b, ln2_g, ln2_b, W2, b2, v_th):
    f32 = jnp.float32
    scale = (jnp.mean(jnp.abs(W1)) + 1e-8).reshape(1, 1)

    Wq = pl.pallas_call(
        _quant_kernel,
        grid=(8,),
        in_specs=[pl.BlockSpec((HID_F // 8, IN_F), lambda i: (i, 0)),
                  pl.BlockSpec((1, 1), lambda i: (0, 0),
                               memory_space=pltpu.SMEM)],
        out_specs=pl.BlockSpec((HID_F // 8, IN_F), lambda i: (i, 0)),
        out_shape=jax.ShapeDtypeStruct((HID_F, IN_F), f32),
    )(W1, scale)

    b1r = b1.reshape(1, HID_F)
    g1r = ln1_g.reshape(1, HID_F)
    be1r = ln1_b.reshape(1, HID_F)
    g2r = ln2_g.reshape(1, HID_F)
    be2r = ln2_b.reshape(1, HID_F)
    vec = lambda: pl.BlockSpec((1, HID_F), lambda i: (0, 0))

    nb = N_TOK // T1
    reservoir, deep = pl.pallas_call(
        _fwd1_kernel,
        grid=(nb + 1,),
        in_specs=[pl.BlockSpec((T1, IN_F), lambda i: (jnp.minimum(i, nb - 1), 0)),
                  pl.BlockSpec((HID_F, IN_F), lambda i: (0, 0)),
                  vec(), vec(), vec(), vec(), vec()],
        out_specs=[pl.BlockSpec((T1, HID_F),
                                lambda i: (jnp.minimum(i, nb - 1), 0)),
                   pl.BlockSpec((T1, HID_F),
                                lambda i: (jnp.maximum(i - 1, 0), 0))],
        out_shape=[jax.ShapeDtypeStruct((N_TOK, HID_F), f32),
                   jax.ShapeDtypeStruct((N_TOK, HID_F), f32)],
        scratch_shapes=[pltpu.VMEM((2, T1, HID_F), f32)],
    )(x_input, Wq, b1r, g1r, be1r, g2r, be2r)

    b2r = b2.reshape(1, OUT_F)
    vthr = v_th.reshape(1, OUT_F)
    out, spikes, membrane, spike_prob = pl.pallas_call(
        _fwd2_kernel,
        grid=(N_TOK // T2,),
        in_specs=[pl.BlockSpec((T2, HID_F), lambda i: (i, 0)),
                  pl.BlockSpec((OUT_F, HID_F), lambda i: (0, 0)),
                  pl.BlockSpec((1, OUT_F), lambda i: (0, 0)),
                  pl.BlockSpec((1, OUT_F), lambda i: (0, 0))],
        out_specs=[pl.BlockSpec((T2, OUT_F), lambda i: (i, 0))] * 4,
        out_shape=[jax.ShapeDtypeStruct((N_TOK, OUT_F), f32)] * 4,
    )(deep, W2, b2r, vthr)

    return out, spikes, reservoir, deep, membrane, spike_prob


# speedup vs baseline: 1.0408x; 1.0408x over previous
import functools

import jax
import jax.numpy as jnp
from jax.experimental import pallas as pl
from jax.experimental.pallas import tpu as pltpu

IN_F = 2048
HID_F = 4096
OUT_F = 2048
N_TOK = 8192
K_TOP = max(1, int(HID_F * 0.15))

T1 = 128
T2 = 256


def _quant_kernel(w_ref, s_ref, o_ref):
    s = s_ref[0, 0]
    w = w_ref[...]
    o_ref[...] = jnp.round(jnp.clip(w / s, -1.0, 1.0)) * s


def _sortable_i32(z):
    m = jax.lax.bitcast_convert_type(z, jnp.int32)
    neg = jnp.bitwise_xor(jnp.bitwise_not(m), jnp.int32(-2147483648))
    return jnp.where(m >= 0, m, neg)


def _topk_mask_deep(z, deep_ref):
    mz = jnp.mean(z, axis=1, keepdims=True)
    sz = jnp.sqrt(jnp.maximum(
        jnp.mean(z * z, axis=1, keepdims=True) - mz * mz, 0.0))
    sz = sz * 1.00001 + 1e-30
    lo0 = mz - 0.43 * sz
    hi0 = mz + 2.50 * sz

    def vbody(_, carry):
        lo, hi, clo, chi = carry
        w = hi - lo
        denom = jnp.maximum((clo - chi).astype(jnp.float32), 1.0)
        frac = (clo - K_TOP).astype(jnp.float32) / denom
        mid = lo + w * jnp.clip(frac, 0.04, 0.96)
        cnt = jnp.sum((z >= mid).astype(jnp.int32), axis=1, keepdims=True)
        ge = cnt >= K_TOP
        eq = cnt == K_TOP
        lo = jnp.where(ge, mid, lo)
        clo = jnp.where(ge, cnt, clo)
        hi = jnp.where(eq, mid, jnp.where(ge, hi, mid))
        chi = jnp.where(eq, cnt, jnp.where(ge, chi, cnt))
        return lo, hi, clo, chi

    t_blk = z.shape[0]
    c_lo0 = jnp.full((t_blk, 1), 639, dtype=jnp.int32)
    c_hi0 = jnp.full((t_blk, 1), 564, dtype=jnp.int32)
    carry = (lo0, hi0, c_lo0, c_hi0)
    for _ in range(8):
        carry = vbody(0, carry)
    lof, hif, _, _ = carry

    s = _sortable_i32(z)
    klo0 = _sortable_i32(lof)
    khi0 = _sortable_i32(hif)

    def kcond(carry):
        i, klo, khi = carry
        return jnp.logical_and(i < 32, jnp.any(khi > klo + 1))

    def kpass(klo, khi):
        mid = jnp.bitwise_and(klo, khi) + jnp.right_shift(
            jnp.bitwise_xor(klo, khi), 1)
        cnt = jnp.sum((s >= mid).astype(jnp.int32), axis=1, keepdims=True)
        ge = cnt >= K_TOP
        eq = cnt == K_TOP
        klo = jnp.where(ge, mid, klo)
        khi = jnp.where(eq, mid + 1, jnp.where(ge, khi, mid))
        return klo, khi

    def kbody(carry):
        i, klo, khi = carry
        klo, khi = kpass(klo, khi)
        klo, khi = kpass(klo, khi)
        return i + 2, klo, khi

    for _ in range(8):
        klo0, khi0 = kpass(klo0, khi0)
    _, klo, _ = jax.lax.while_loop(kcond, kbody, (0, klo0, khi0))
    mask = (s >= klo).astype(jnp.float32)
    deep_ref[...] = z * mask


def _fwd1_kernel(x_ref, w_ref, b1_ref, g1_ref, be1_ref, g2_ref, be2_ref,
                 res_ref, deep_ref, zscr_ref):
    i = pl.program_id(0)
    p = jax.lax.rem(i, 2)

    @pl.when(i == 0)
    def _init():
        zscr_ref[...] = jnp.zeros_like(zscr_ref)

    x = x_ref[...]
    h = jax.lax.dot_general(x, w_ref[...], (((1,), (1,)), ((), ())),
                            preferred_element_type=jnp.float32)
    h = h + b1_ref[...]
    h = jnp.maximum(h, 0.0)
    mu = jnp.mean(h, axis=1, keepdims=True)
    d = h - mu
    var = jnp.mean(d * d, axis=1, keepdims=True)
    res = d / jnp.sqrt(var + 1e-5) * g1_ref[...] + be1_ref[...]
    mu2 = jnp.mean(res, axis=1, keepdims=True)
    d2 = res - mu2
    var2 = jnp.mean(d2 * d2, axis=1, keepdims=True)
    z = d2 / jnp.sqrt(var2 + 1e-5) * g2_ref[...] + be2_ref[...]

    z_prev = zscr_ref[pl.ds(1 - p, 1), :, :][0]
    _topk_mask_deep(z_prev, deep_ref)

    res_ref[...] = res
    zscr_ref[pl.ds(p, 1), :, :] = z[None]


def _fwd2_kernel(d_ref, w_ref, b2_ref, vth_ref,
                 out_ref, spk_ref, mem_ref, sp_ref):
    mem = jax.lax.dot_general(d_ref[...], w_ref[...], (((1,), (1,)), ((), ())),
                              preferred_element_type=jnp.float32)
    mem = mem + b2_ref[...]
    vth = vth_ref[...]
    sp = jax.nn.sigmoid((mem - vth) / 0.1)
    hard = (mem > vth).astype(jnp.float32)
    mem_ref[...] = mem
    sp_ref[...] = sp
    spk_ref[...] = hard
    out_ref[...] = mem * hard


@jax.jit
def kernel(x_input, W1, b1, ln1_g, ln1_b, ln2_g, ln2_b, W2, b2, v_th):
    f32 = jnp.float32
    scale = (jnp.mean(jnp.abs(W1)) + 1e-8).reshape(1, 1)

    Wq = pl.pallas_call(
        _quant_kernel,
        grid=(8,),
        in_specs=[pl.BlockSpec((HID_F // 8, IN_F), lambda i: (i, 0)),
                  pl.BlockSpec((1, 1), lambda i: (0, 0),
                               memory_space=pltpu.SMEM)],
        out_specs=pl.BlockSpec((HID_F // 8, IN_F), lambda i: (i, 0)),
        out_shape=jax.ShapeDtypeStruct((HID_F, IN_F), f32),
    )(W1, scale)

    b1r = b1.reshape(1, HID_F)
    g1r = ln1_g.reshape(1, HID_F)
    be1r = ln1_b.reshape(1, HID_F)
    g2r = ln2_g.reshape(1, HID_F)
    be2r = ln2_b.reshape(1, HID_F)
    vec = lambda: pl.BlockSpec((1, HID_F), lambda i: (0, 0))

    nb = N_TOK // T1
    reservoir, deep = pl.pallas_call(
        _fwd1_kernel,
        grid=(nb + 1,),
        in_specs=[pl.BlockSpec((T1, IN_F), lambda i: (jnp.minimum(i, nb - 1), 0)),
                  pl.BlockSpec((HID_F, IN_F), lambda i: (0, 0)),
                  vec(), vec(), vec(), vec(), vec()],
        out_specs=[pl.BlockSpec((T1, HID_F),
                                lambda i: (jnp.minimum(i, nb - 1), 0)),
                   pl.BlockSpec((T1, HID_F),
                                lambda i: (jnp.maximum(i - 1, 0), 0))],
        out_shape=[jax.ShapeDtypeStruct((N_TOK, HID_F), f32),
                   jax.ShapeDtypeStruct((N_TOK, HID_F), f32)],
        scratch_shapes=[pltpu.VMEM((2, T1, HID_F), f32)],
    )(x_input, Wq, b1r, g1r, be1r, g2r, be2r)

    b2r = b2.reshape(1, OUT_F)
    vthr = v_th.reshape(1, OUT_F)
    out, spikes, membrane, spike_prob = pl.pallas_call(
        _fwd2_kernel,
        grid=(N_TOK // T2,),
        in_specs=[pl.BlockSpec((T2, HID_F), lambda i: (i, 0)),
                  pl.BlockSpec((OUT_F, HID_F), lambda i: (0, 0)),
                  pl.BlockSpec((1, OUT_F), lambda i: (0, 0)),
                  pl.BlockSpec((1, OUT_F), lambda i: (0, 0))],
        out_specs=[pl.BlockSpec((T2, OUT_F), lambda i: (i, 0))] * 4,
        out_shape=[jax.ShapeDtypeStruct((N_TOK, OUT_F), f32)] * 4,
    )(deep, W2, b2r, vthr)

    return out, spikes, reservoir, deep, membrane, spike_prob
